# baseline (speedup 1.0000x reference)
import functools

import jax
import jax.numpy as jnp
from jax import lax
from jax.experimental import pallas as pl
from jax.experimental.pallas import tpu as pltpu
from jax.experimental.pallas import tpu_sc as plsc

VOCAB = 1000000
DIM = 64
B = 4096
T = 200

NTOK = B * T
NC = 2
NS = 16
NW = NC * NS
PER_W = NTOK // NW
CHUNK = 128
NCHUNK = PER_W // CHUNK
NBUF = 4
NSTEP = NCHUNK // NBUF - 1

NPAIR = VOCAB // 2
WFULL = VOCAB // CHUNK
WTAIL = VOCAB - WFULL * CHUNK
WITER = WFULL // NW + 1

_mesh = plsc.VectorSubcoreMesh(core_axis_name="c", subcore_axis_name="s")


@functools.partial(
    pl.kernel,
    out_type=jax.ShapeDtypeStruct((NPAIR, 2 * DIM), jnp.float32),
    mesh=_mesh,
    scratch_types=(
        [pltpu.VMEM((DIM, CHUNK), jnp.float32) for _ in range(2)]
        + [pltpu.VMEM((DIM, 2 * DIM), jnp.float32) for _ in range(2)]
        + [pltpu.SemaphoreType.DMA for _ in range(4)]
    ),
    compiler_params=pltpu.CompilerParams(
        use_tc_tiling_on_sc=True, needs_layout_passes=False
    ),
)
def _widen(tt_hbm, tail_hbm, pairs_hbm, in0, in1, out0, out1, si0, si1, sw0, sw1):
    ins = (in0, in1)
    outs = (out0, out1)
    sis = (si0, si1)
    sws = (sw0, sw1)
    wid = lax.axis_index("s") * NC + lax.axis_index("c")

    @pl.when(wid == 0)
    def _tail():
        pltpu.sync_copy(tail_hbm, pairs_hbm.at[pl.ds(WFULL * DIM, WTAIL // 2)])

    def stage(c, k):
        pltpu.async_copy(
            tt_hbm.at[:, pl.ds(c * CHUNK, CHUNK)], ins[k], sis[k]
        )

    def wait_stage(k):
        pltpu.make_async_copy(
            tt_hbm.at[:, pl.ds(0, CHUNK)], ins[k], sis[k]
        ).wait()

    def write(c, k):
        pltpu.async_copy(outs[k], pairs_hbm.at[pl.ds(c * DIM, DIM)], sws[k])

    def wait_write(k):
        pltpu.make_async_copy(outs[k], pairs_hbm.at[pl.ds(0, DIM)], sws[k]).wait()

    def transpose(k):
        iot = lax.iota(jnp.int32, 16)
        for u in range(DIM):
            for q in range(8):
                f = (16 * q) % DIM + iot
                col = jnp.full((16,), 2 * u + (1 if q >= 4 else 0), jnp.int32)
                vals = plsc.load_gather(ins[k], [f, col])
                outs[k][u, pl.ds(16 * q, 16)] = vals

    NFULL = WFULL // NW
    stage(wid, 0)

    def step(i, carry):
        c = carry
        for k in range(2):
            cn = c + NW

            @pl.when(2 * i + k < NFULL - 1)
            def _():
                stage(cn, 1 - k)

            wait_stage(k)
            transpose(k)
            write(c, k)
            wait_write(k)
            c = cn
        return c

    lax.fori_loop(0, NFULL // 2, step, wid)

    @pl.when(wid < WFULL - NFULL * NW)
    def _extra():
        ce = NFULL * NW + wid
        stage(ce, 0)
        wait_stage(0)
        transpose(0)
        write(ce, 0)
        wait_write(0)


@functools.partial(
    pl.kernel,
    out_type=jax.ShapeDtypeStruct((NTOK, 2 * DIM), jnp.float32),
    mesh=_mesh,
    scratch_types=(
        [pltpu.VMEM((NCHUNK, CHUNK), jnp.int32)]
        + [pltpu.VMEM((CHUNK, DIM), jnp.float32) for _ in range(NBUF)]
        + [pltpu.SemaphoreType.DMA for _ in range(2 * NBUF)]
    ),
    compiler_params=pltpu.CompilerParams(
        use_tc_tiling_on_sc=False, needs_layout_passes=False
    ),
)
def _gather(x_hbm, table_hbm, out_hbm, idx_all, *bufs_and_sems):
    rows = bufs_and_sems[:NBUF]
    sem_g = bufs_and_sems[NBUF : 2 * NBUF]
    sem_w = bufs_and_sems[2 * NBUF : 3 * NBUF]

    wid = lax.axis_index("s") * NC + lax.axis_index("c")
    base = wid * PER_W

    pltpu.sync_copy(x_hbm.at[pl.ds(wid * NCHUNK, NCHUNK)], idx_all)

    def gather(c, k):
        pltpu.async_copy(table_hbm.at[idx_all.at[c]], rows[k], sem_g[k])

    def wait_gather(k):
        pltpu.make_async_copy(
            table_hbm.at[pl.ds(0, CHUNK)], rows[k], sem_g[k]
        ).wait()

    def write(c, k):
        pltpu.async_copy(
            rows[k],
            out_hbm.at[pl.ds(base + c * CHUNK, CHUNK), pl.ds(0, DIM)],
            sem_w[k],
        )

    def wait_write(k):
        pltpu.make_async_copy(
            rows[k], out_hbm.at[pl.ds(0, CHUNK), pl.ds(0, DIM)], sem_w[k]
        ).wait()

    def fixup(c, k):
        zeros16 = jnp.zeros((16,), jnp.float32)
        for g in range(CHUNK // 16):
            iv = idx_all[c, pl.ds(g * 16, 16)]
            m = iv == 0
            npad = jnp.sum(m.astype(jnp.int32))

            @pl.when(npad > 0)
            def _():
                rid = g * 16 + lax.iota(jnp.int32, 16)
                for d in range(DIM):
                    plsc.store_scatter(
                        rows[k],
                        [rid, jnp.full((16,), d, jnp.int32)],
                        zeros16,
                        mask=m,
                    )

    for k in range(NBUF):
        gather(k, k)

    def step(j, carry):
        for k in range(NBUF):
            c = j * NBUF + k
            wait_gather(k)
            fixup(c, k)
            write(c, k)
            wait_write(k)
            gather(c + NBUF, k)
        return carry

    lax.fori_loop(0, NSTEP, step, 0)

    for k in range(NBUF):
        c = NSTEP * NBUF + k
        wait_gather(k)
        fixup(c, k)
        write(c, k)
    for k in range(NBUF):
        wait_write(k)


def kernel(x, table):
    tail_pairs = table[WFULL * CHUNK :].reshape(WTAIL // 2, 2 * DIM)
    pairs = _widen(table.T, tail_pairs)
    table_lin = pairs.reshape(VOCAB, DIM)
    out = _gather(x.reshape(NTOK // CHUNK, CHUNK), table_lin)
    out64 = lax.slice(out, (0, 0), (NTOK, DIM))
    return out64.reshape(B, T, DIM)


# speedup vs baseline: 2.4737x; 2.4737x over previous
import functools

import jax
import jax.numpy as jnp
from jax import lax
from jax.experimental import pallas as pl
from jax.experimental.pallas import tpu as pltpu
from jax.experimental.pallas import tpu_sc as plsc

VOCAB = 1000000
DIM = 64
B = 4096
T = 200

NTOK = B * T
NC = 2
NS = 16
NW = NC * NS
PER_W = NTOK // NW
CHUNK = 128
NCHUNK = PER_W // CHUNK
NBUF = 4
NSTEP = NCHUNK // NBUF - 1

_mesh = plsc.VectorSubcoreMesh(core_axis_name="c", subcore_axis_name="s")

VB = 2048


def _widen_body(tt_ref, out_ref):
    t = tt_ref[...].T
    out_ref[...] = jnp.concatenate(
        [t, jnp.zeros((VB, DIM), jnp.float32)], axis=1
    )


_widen = pl.pallas_call(
    _widen_body,
    grid=(pl.cdiv(VOCAB, VB),),
    in_specs=[pl.BlockSpec((DIM, VB), lambda i: (0, i))],
    out_specs=pl.BlockSpec((VB, 2 * DIM), lambda i: (i, 0)),
    out_shape=jax.ShapeDtypeStruct((VOCAB, 2 * DIM), jnp.float32),
)


@functools.partial(
    pl.kernel,
    out_type=jax.ShapeDtypeStruct((NTOK, 2 * DIM), jnp.float32),
    mesh=_mesh,
    scratch_types=(
        [pltpu.VMEM((NCHUNK, CHUNK), jnp.int32)]
        + [pltpu.VMEM((CHUNK, DIM), jnp.float32) for _ in range(NBUF)]
        + [pltpu.SemaphoreType.DMA for _ in range(2 * NBUF)]
    ),
    compiler_params=pltpu.CompilerParams(
        use_tc_tiling_on_sc=False, needs_layout_passes=False
    ),
)
def _gather(x_hbm, table_hbm, out_hbm, idx_all, *bufs_and_sems):
    rows = bufs_and_sems[:NBUF]
    sem_g = bufs_and_sems[NBUF : 2 * NBUF]
    sem_w = bufs_and_sems[2 * NBUF : 3 * NBUF]

    wid = lax.axis_index("s") * NC + lax.axis_index("c")
    base = wid * PER_W

    pltpu.sync_copy(x_hbm.at[pl.ds(wid * NCHUNK, NCHUNK)], idx_all)

    def gather(c, k):
        pltpu.async_copy(table_hbm.at[idx_all.at[c]], rows[k], sem_g[k])

    def wait_gather(k):
        pltpu.make_async_copy(
            table_hbm.at[pl.ds(0, CHUNK)], rows[k], sem_g[k]
        ).wait()

    def write(c, k):
        pltpu.async_copy(
            rows[k],
            out_hbm.at[pl.ds(base + c * CHUNK, CHUNK), pl.ds(0, DIM)],
            sem_w[k],
        )

    def wait_write(k):
        pltpu.make_async_copy(
            rows[k], out_hbm.at[pl.ds(0, CHUNK), pl.ds(0, DIM)], sem_w[k]
        ).wait()

    def fixup(c, k):
        zeros16 = jnp.zeros((16,), jnp.float32)
        for g in range(CHUNK // 16):
            iv = idx_all[c, pl.ds(g * 16, 16)]
            m = iv == 0
            npad = jnp.sum(m.astype(jnp.int32))

            @pl.when(npad > 0)
            def _():
                rid = g * 16 + lax.iota(jnp.int32, 16)
                for d in range(DIM):
                    plsc.store_scatter(
                        rows[k],
                        [rid, jnp.full((16,), d, jnp.int32)],
                        zeros16,
                        mask=m,
                    )

    for k in range(NBUF):
        gather(k, k)

    def step(j, carry):
        for k in range(NBUF):
            c = j * NBUF + k
            wait_gather(k)
            fixup(c, k)
            write(c, k)
            wait_write(k)
            gather(c + NBUF, k)
        return carry

    lax.fori_loop(0, NSTEP, step, 0)

    for k in range(NBUF):
        c = NSTEP * NBUF + k
        wait_gather(k)
        fixup(c, k)
        write(c, k)
    for k in range(NBUF):
        wait_write(k)


def kernel(x, table):
    table_wide = _widen(table.T)
    table_lin = table_wide.reshape(2 * VOCAB, DIM)
    x2 = (x * 2).reshape(NTOK // CHUNK, CHUNK)
    out = _gather(x2, table_lin)
    out64 = lax.slice(out, (0, 0), (NTOK, DIM))
    return out64.reshape(B, T, DIM)


# speedup vs baseline: 2.4815x; 1.0031x over previous
import functools

import jax
import jax.numpy as jnp
from jax import lax
from jax.experimental import pallas as pl
from jax.experimental.pallas import tpu as pltpu
from jax.experimental.pallas import tpu_sc as plsc

VOCAB = 1000000
DIM = 64
B = 4096
T = 200

NTOK = B * T
NC = 2
NS = 16
NW = NC * NS
PER_W = NTOK // NW
CHUNK = 128
NCHUNK = PER_W // CHUNK
NBUF = 4
NSTEP = NCHUNK // NBUF - 1

_mesh = plsc.VectorSubcoreMesh(core_axis_name="c", subcore_axis_name="s")

VB = 2048


def _widen_body(tt_ref, out_ref):
    out_ref[:, 0:DIM] = tt_ref[...].T


_widen = pl.pallas_call(
    _widen_body,
    grid=(pl.cdiv(VOCAB, VB),),
    in_specs=[pl.BlockSpec((DIM, VB), lambda i: (0, i))],
    out_specs=pl.BlockSpec((VB, 2 * DIM), lambda i: (i, 0)),
    out_shape=jax.ShapeDtypeStruct((VOCAB, 2 * DIM), jnp.float32),
)


@functools.partial(
    pl.kernel,
    out_type=jax.ShapeDtypeStruct((NTOK, 2 * DIM), jnp.float32),
    mesh=_mesh,
    scratch_types=(
        [pltpu.VMEM((NCHUNK, CHUNK), jnp.int32)]
        + [pltpu.VMEM((CHUNK, DIM), jnp.float32) for _ in range(NBUF)]
        + [pltpu.SemaphoreType.DMA for _ in range(2 * NBUF)]
    ),
    compiler_params=pltpu.CompilerParams(
        use_tc_tiling_on_sc=False, needs_layout_passes=False
    ),
)
def _gather(x_hbm, table_hbm, out_hbm, idx_all, *bufs_and_sems):
    rows = bufs_and_sems[:NBUF]
    sem_g = bufs_and_sems[NBUF : 2 * NBUF]
    sem_w = bufs_and_sems[2 * NBUF : 3 * NBUF]

    wid = lax.axis_index("s") * NC + lax.axis_index("c")
    base = wid * PER_W

    pltpu.sync_copy(x_hbm.at[pl.ds(wid * NCHUNK, NCHUNK)], idx_all)

    def gather(c, k):
        pltpu.async_copy(table_hbm.at[idx_all.at[c]], rows[k], sem_g[k])

    def wait_gather(k):
        pltpu.make_async_copy(
            table_hbm.at[pl.ds(0, CHUNK)], rows[k], sem_g[k]
        ).wait()

    def write(c, k):
        pltpu.async_copy(
            rows[k],
            out_hbm.at[pl.ds(base + c * CHUNK, CHUNK), pl.ds(0, DIM)],
            sem_w[k],
        )

    def wait_write(k):
        pltpu.make_async_copy(
            rows[k], out_hbm.at[pl.ds(0, CHUNK), pl.ds(0, DIM)], sem_w[k]
        ).wait()

    def fixup(c, k):
        zeros16 = jnp.zeros((16,), jnp.float32)
        for g in range(CHUNK // 16):
            iv = idx_all[c, pl.ds(g * 16, 16)]
            m = iv == 0
            npad = jnp.sum(m.astype(jnp.int32))

            @pl.when(npad > 0)
            def _():
                rid = g * 16 + lax.iota(jnp.int32, 16)
                for d in range(DIM):
                    plsc.store_scatter(
                        rows[k],
                        [rid, jnp.full((16,), d, jnp.int32)],
                        zeros16,
                        mask=m,
                    )

    for k in range(NBUF):
        gather(k, k)

    def step(j, carry):
        for k in range(NBUF):
            c = j * NBUF + k
            wait_gather(k)
            fixup(c, k)
            write(c, k)
            wait_write(k)
            gather(c + NBUF, k)
        return carry

    lax.fori_loop(0, NSTEP, step, 0)

    for k in range(NBUF):
        c = NSTEP * NBUF + k
        wait_gather(k)
        fixup(c, k)
        write(c, k)
    for k in range(NBUF):
        wait_write(k)


def kernel(x, table):
    table_wide = _widen(table.T)
    table_lin = table_wide.reshape(2 * VOCAB, DIM)
    x2 = (x * 2).reshape(NTOK // CHUNK, CHUNK)
    out = _gather(x2, table_lin)
    out64 = lax.slice(out, (0, 0), (NTOK, DIM))
    return out64.reshape(B, T, DIM)
